# baseline (speedup 1.0000x reference)
import functools
import jax
import jax.numpy as jnp
from jax import lax
from jax.experimental import pallas as pl
from jax.experimental.pallas import tpu as pltpu, tpu_sc as plsc

NC, NS, LANES = 2, 16, 16
NW = NC * NS

C = 128
NBUF = 4


def _sc_body(num_fields, vocab, dim, nrounds, tab_hbm, idx_hbm, out_hbm,
             blk, idxb, rows, acc2, gsem, outsem):
    cid = lax.axis_index("c")
    sid = lax.axis_index("s")
    wid = sid * NC + cid
    tw = nrounds * C
    base = wid * tw
    lanes_iota = lax.iota(jnp.int32, LANES)

    def round_fn(r, _):
        tbase = base + r * C
        p = r % 2

        @pl.when(r >= 2)
        def _():
            pltpu.make_async_copy(
                acc2.at[p], out_hbm.at[pl.ds(tbase, C)], outsem.at[p]).wait()

        pltpu.sync_copy(idx_hbm.at[pl.ds(tbase, C)], blk)

        def fmt_fn(i, _):
            col = jnp.full((LANES,), 0, jnp.int32) + i
            off = i * vocab

            def fmt_j(j, _):
                row = j * LANES + lanes_iota
                vals = plsc.load_gather(blk, [row, col])
                idxb[i, pl.ds(j * LANES, LANES)] = vals + off
                return 0
            lax.fori_loop(0, C // LANES, fmt_j, 0, unroll=True)
            return 0
        lax.fori_loop(0, num_fields, fmt_fn, 0)

        d0 = pltpu.async_copy(tab_hbm.at[idxb.at[0]], acc2.at[p],
                              gsem.at[NBUF])
        for b in range(NBUF):
            pltpu.async_copy(tab_hbm.at[idxb.at[1 + b]], rows.at[b],
                             gsem.at[b])
        d0.wait()

        def field_fn(i, _):
            b = (i - 1) % NBUF
            pltpu.make_async_copy(tab_hbm.at[idxb.at[i]], rows.at[b],
                                  gsem.at[b]).wait()

            def add_fn(t, _):
                lo = pl.ds(0, LANES)
                hi = pl.ds(LANES, LANES)
                acc2[p, t, lo] = acc2[p, t, lo] + rows[b, t, lo]
                acc2[p, t, hi] = acc2[p, t, hi] + rows[b, t, hi]
                return 0
            lax.fori_loop(0, C, add_fn, 0, unroll=4)

            @pl.when(i + NBUF < num_fields)
            def _():
                pltpu.async_copy(tab_hbm.at[idxb.at[i + NBUF]], rows.at[b],
                                 gsem.at[b])
            return 0
        lax.fori_loop(1, num_fields, field_fn, 0)

        pltpu.async_copy(acc2.at[p], out_hbm.at[pl.ds(tbase, C)],
                         outsem.at[p])
        return 0

    lax.fori_loop(0, nrounds, round_fn, 0)

    for p in range(2):
        pltpu.make_async_copy(
            acc2.at[p], out_hbm.at[pl.ds(base, C)], outsem.at[p]).wait()


def kernel(x, tables):
    B, L, F = x.shape
    _, V, D = tables.shape
    T = B * L
    assert T % (NW * C) == 0
    nrounds = T // (NW * C)
    assert nrounds >= 2 and nrounds % 2 == 0

    idx_flat = x.reshape(T, F)
    tab_flat = tables.reshape(F * V, D)

    run = pl.kernel(
        functools.partial(_sc_body, F, V, D, nrounds),
        out_type=jax.ShapeDtypeStruct((T, D), jnp.float32),
        mesh=plsc.VectorSubcoreMesh(core_axis_name="c", subcore_axis_name="s",
                                    num_cores=NC, num_subcores=NS),
        scratch_types=[
            pltpu.VMEM((C, F), jnp.int32),
            pltpu.VMEM((F, C), jnp.int32),
            pltpu.VMEM((NBUF, C, D), jnp.float32),
            pltpu.VMEM((2, C, D), jnp.float32),
            pltpu.SemaphoreType.DMA((NBUF + 1,)),
            pltpu.SemaphoreType.DMA((2,)),
        ],
        compiler_params=pltpu.CompilerParams(use_tc_tiling_on_sc=False,
                                             needs_layout_passes=False),
    )
    out = run(tab_flat, idx_flat)
    return out.reshape(B, L, D)


# speedup vs baseline: 1.9185x; 1.9185x over previous
import functools
import jax
import jax.numpy as jnp
from jax import lax
from jax.experimental import pallas as pl
from jax.experimental.pallas import tpu as pltpu, tpu_sc as plsc

NC, NS, LANES = 2, 16, 16
NW = NC * NS

CT = 64
NVEC = 13
ROWS = CT * 26


def _sc_body(num_fields, vocab, dim, nrounds, tab_hbm, idx_hbm, out_hbm,
             idxb, rows2, outbuf, offvec, gsem, isem, outsem):
    cid = lax.axis_index("c")
    sid = lax.axis_index("s")
    wid = sid * NC + cid
    tw = nrounds * CT
    base = wid * tw
    ibase = wid * nrounds * NVEC
    lanes = lax.iota(jnp.int32, LANES)
    lo = pl.ds(0, LANES)
    hi = pl.ds(LANES, LANES)

    for j in range(NVEC):
        v = lax.rem(j * LANES + lanes, num_fields) * vocab
        offvec[pl.ds(j * LANES, LANES)] = v

    def stage_idx(r, sync):
        src = idx_hbm.at[pl.ds(ibase + r * NVEC, NVEC)]
        if sync:
            pltpu.sync_copy(src, idxb.at[r % 2])
        else:
            pltpu.async_copy(src, idxb.at[r % 2], isem.at[r % 2])

    def wait_idx(r):
        pltpu.make_async_copy(idx_hbm.at[pl.ds(ibase, NVEC)],
                              idxb.at[r % 2], isem.at[r % 2]).wait()

    def offset_add(r):
        p = r % 2
        for j in range(2 * NVEC * 4):
            row, sl = j // 8, pl.ds((j % 8) * LANES, LANES)
            osl = pl.ds((j % NVEC) * LANES, LANES)
            idxb[p, row, sl] = idxb[p, row, sl] + offvec[osl]

    def fire_gathers(r):
        p = r % 2
        for g in range(NVEC):
            pltpu.async_copy(tab_hbm.at[idxb.at[p, g]],
                             rows2.at[p, pl.ds(g * 128, 128)], gsem.at[p])

    def drain_gathers(r):
        p = r % 2
        for g in range(NVEC):
            pltpu.make_async_copy(tab_hbm.at[idxb.at[p, g]],
                                  rows2.at[p, pl.ds(g * 128, 128)],
                                  gsem.at[p]).wait()

    def wait_out(r):
        p = r % 2
        pltpu.make_async_copy(outbuf.at[p], out_hbm.at[pl.ds(base, CT)],
                              outsem.at[p]).wait()

    stage_idx(0, True)
    offset_add(0)
    fire_gathers(0)
    stage_idx(1, False)

    def round_fn(r, _):
        p = r % 2

        @pl.when(r + 1 < nrounds)
        def _():
            wait_idx(r + 1)
            offset_add(r + 1)
            fire_gathers(r + 1)

        drain_gathers(r)

        @pl.when(r + 2 < nrounds)
        def _():
            stage_idx(r + 2, False)

        @pl.when(r >= 2)
        def _():
            wait_out(r)

        def acc_fn(t, _):
            s = t * num_fields
            a = rows2[p, s, lo]
            b = rows2[p, s, hi]
            for k in range(1, num_fields):
                a = a + rows2[p, s + k, lo]
                b = b + rows2[p, s + k, hi]
            outbuf[p, t, lo] = a
            outbuf[p, t, hi] = b
            return 0
        lax.fori_loop(0, CT, acc_fn, 0, unroll=2)

        pltpu.async_copy(outbuf.at[p],
                         out_hbm.at[pl.ds(base + r * CT, CT)], outsem.at[p])
        return 0

    lax.fori_loop(0, nrounds, round_fn, 0)

    for p in range(2):
        pltpu.make_async_copy(outbuf.at[p], out_hbm.at[pl.ds(base, CT)],
                              outsem.at[p]).wait()


def kernel(x, tables):
    B, L, F = x.shape
    _, V, D = tables.shape
    T = B * L
    assert F == 26 and D == 32
    assert T % (NW * CT) == 0
    nrounds = T // (NW * CT)
    assert nrounds >= 3

    idx_flat = x.reshape(T * F // 128, 128)
    tab_flat = tables.reshape(F * V, D)

    run = pl.kernel(
        functools.partial(_sc_body, F, V, D, nrounds),
        out_type=jax.ShapeDtypeStruct((T, D), jnp.float32),
        mesh=plsc.VectorSubcoreMesh(core_axis_name="c", subcore_axis_name="s",
                                    num_cores=NC, num_subcores=NS),
        scratch_types=[
            pltpu.VMEM((2, NVEC, 128), jnp.int32),
            pltpu.VMEM((2, ROWS, D), jnp.float32),
            pltpu.VMEM((2, CT, D), jnp.float32),
            pltpu.VMEM((NVEC * LANES,), jnp.int32),
            pltpu.SemaphoreType.DMA((2,)),
            pltpu.SemaphoreType.DMA((2,)),
            pltpu.SemaphoreType.DMA((2,)),
        ],
        compiler_params=pltpu.CompilerParams(use_tc_tiling_on_sc=False),
    )
    out = run(tab_flat, idx_flat)
    return out.reshape(B, L, D)


# speedup vs baseline: 3.0229x; 1.5756x over previous
import functools
import jax
import jax.numpy as jnp
from jax import lax
from jax.experimental import pallas as pl
from jax.experimental.pallas import tpu as pltpu, tpu_sc as plsc

NC, NS, LANES = 2, 16, 16
NW = NC * NS

CT = 64
NVEC = 13
ROWS = CT * 26


def _sc_body(num_fields, vocab, dim, nrounds, tab_hbm, idx_hbm, out_hbm,
             idxb, rows2, outbuf, offvec, gsem, isem, outsem):
    cid = lax.axis_index("c")
    sid = lax.axis_index("s")
    wid = sid * NC + cid
    tw = nrounds * CT
    base = wid * tw
    ibase = wid * nrounds * NVEC
    lanes = lax.iota(jnp.int32, LANES)
    lo = pl.ds(0, LANES)
    hi = pl.ds(LANES, LANES)

    for j in range(NVEC):
        v = lax.rem(j * LANES + lanes, num_fields)
        offvec[pl.ds(j * LANES, LANES)] = v

    def stage_idx(r, sync):
        src = idx_hbm.at[pl.ds(ibase + r * NVEC, NVEC)]
        if sync:
            pltpu.sync_copy(src, idxb.at[r % 2])
        else:
            pltpu.async_copy(src, idxb.at[r % 2], isem.at[r % 2])

    def wait_idx(r):
        pltpu.make_async_copy(idx_hbm.at[pl.ds(ibase, NVEC)],
                              idxb.at[r % 2], isem.at[r % 2]).wait()

    def offset_add(r):
        p = r % 2
        for j in range(2 * NVEC * 4):
            row, sl = j // 8, pl.ds((j % 8) * LANES, LANES)
            osl = pl.ds((j % NVEC) * LANES, LANES)
            idxb[p, row, sl] = idxb[p, row, sl] * 32 + offvec[osl]

    def fire_gathers(r):
        p = r % 2
        for g in range(NVEC):
            pltpu.async_copy(tab_hbm.at[idxb.at[p, g]],
                             rows2.at[p, pl.ds(g * 128, 128)], gsem.at[p])

    def drain_gathers(r):
        p = r % 2
        for g in range(NVEC):
            pltpu.make_async_copy(tab_hbm.at[idxb.at[p, g]],
                                  rows2.at[p, pl.ds(g * 128, 128)],
                                  gsem.at[p]).wait()

    def wait_out(r):
        p = r % 2
        pltpu.make_async_copy(outbuf.at[p], out_hbm.at[pl.ds(base, CT)],
                              outsem.at[p]).wait()

    stage_idx(0, True)
    offset_add(0)
    fire_gathers(0)
    stage_idx(1, False)

    def round_fn(r, _):
        p = r % 2

        @pl.when(r + 1 < nrounds)
        def _():
            wait_idx(r + 1)
            offset_add(r + 1)
            fire_gathers(r + 1)

        drain_gathers(r)

        @pl.when(r + 2 < nrounds)
        def _():
            stage_idx(r + 2, False)

        @pl.when(r >= 2)
        def _():
            wait_out(r)

        def acc_fn(t, _):
            s = t * num_fields
            a = rows2[p, s, lo]
            b = rows2[p, s, hi]
            for k in range(1, num_fields):
                a = a + rows2[p, s + k, lo]
                b = b + rows2[p, s + k, hi]
            outbuf[p, t, lo] = a
            outbuf[p, t, hi] = b
            return 0
        lax.fori_loop(0, CT, acc_fn, 0, unroll=2)

        pltpu.async_copy(outbuf.at[p],
                         out_hbm.at[pl.ds(base + r * CT, CT)], outsem.at[p])
        return 0

    lax.fori_loop(0, nrounds, round_fn, 0)

    for p in range(2):
        pltpu.make_async_copy(outbuf.at[p], out_hbm.at[pl.ds(base, CT)],
                              outsem.at[p]).wait()


def _repack_tables(tables):
    F, V, D = tables.shape
    tab2 = tables.transpose(0, 2, 1).reshape(F * D, V)
    tabp = jnp.pad(tab2, ((0, 1024 - F * D), (0, 0)))
    taby = tabp.reshape(8, 128, V).transpose(2, 0, 1)
    return taby.reshape(V * 32, D)


def kernel(x, tables):
    B, L, F = x.shape
    _, V, D = tables.shape
    T = B * L
    assert F == 26 and D == 32
    assert T % (NW * CT) == 0
    nrounds = T // (NW * CT)
    assert nrounds >= 3

    idx_flat = x.reshape(T * F // 128, 128)
    tab_flat = _repack_tables(tables)

    run = pl.kernel(
        functools.partial(_sc_body, F, V, D, nrounds),
        out_type=jax.ShapeDtypeStruct((T, D), jnp.float32),
        mesh=plsc.VectorSubcoreMesh(core_axis_name="c", subcore_axis_name="s",
                                    num_cores=NC, num_subcores=NS),
        scratch_types=[
            pltpu.VMEM((2, NVEC, 128), jnp.int32),
            pltpu.VMEM((2, ROWS, D), jnp.float32),
            pltpu.VMEM((2, CT, D), jnp.float32),
            pltpu.VMEM((NVEC * LANES,), jnp.int32),
            pltpu.SemaphoreType.DMA((2,)),
            pltpu.SemaphoreType.DMA((2,)),
            pltpu.SemaphoreType.DMA((2,)),
        ],
        compiler_params=pltpu.CompilerParams(use_tc_tiling_on_sc=False),
    )
    out = run(tab_flat, idx_flat)
    return out.reshape(B, L, D)
